# baseline (speedup 1.0000x reference)
import jax
import jax.numpy as jnp
from jax.experimental import pallas as pl
from jax.experimental.pallas import tpu as pltpu

_N = 2048
_H = 32
_R = 256
_C = _N // _R
_PREC = jax.lax.Precision.HIGHEST
_DN_T = (((0,), (0,)), ((), ()))


def _gnn_kernel(adj_hbm, w1_ref, b1_ref, w2_ref, b2_ref, wfc_ref, bfc_ref,
                out_ref, row_ref, dcol_ref, y2_ref, z_ref, ycat_ref, abf_ref,
                stage_ref, sem):
    def start_copy(k, slot):
        pltpu.make_async_copy(adj_hbm.at[pl.ds(k * _R, _R), :],
                              stage_ref.at[slot], sem.at[slot]).start()

    def wait_copy(k, slot):
        pltpu.make_async_copy(adj_hbm.at[pl.ds(k * _R, _R), :],
                              stage_ref.at[slot], sem.at[slot]).wait()

    start_copy(0, 0)
    row_ref[...] = jnp.zeros((1, _N), jnp.float32)

    def p1(k, carry):
        slot = jax.lax.rem(k, 2)

        @pl.when(k + 1 < _C)
        def _():
            start_copy(k + 1, jax.lax.rem(k + 1, 2))

        wait_copy(k, slot)
        af = (stage_ref[slot] > 0).astype(jnp.float32)
        abf_ref[pl.ds(k * _R, _R), :] = af.astype(jnp.bfloat16)
        row_ref[...] += jnp.sum(af, axis=0, keepdims=True)
        return carry

    jax.lax.fori_loop(0, _C, p1, 0)
    dinv_row = 1.0 / jnp.sqrt(row_ref[...] + 1.0)
    dcol_ref[...] = jnp.reshape(dinv_row, (_N, 1))

    row_ref[...] = jnp.zeros((1, _N), jnp.float32)

    def p2(k, carry):
        d = dcol_ref[pl.ds(k * _R, _R), :]
        af = abf_ref[pl.ds(k * _R, _R), :].astype(jnp.float32)
        row_ref[...] += jnp.sum(af * d, axis=0, keepdims=True)
        return carry

    jax.lax.fori_loop(0, _C, p2, 0)

    alpha_row = dinv_row * (row_ref[...] + dinv_row)
    alpha = jnp.reshape(alpha_row, (_N, 1))
    dinv = dcol_ref[...]
    x1 = jax.nn.relu(alpha * w1_ref[...] + b1_ref[...])
    y2 = dinv * jnp.dot(x1, w2_ref[...], precision=_PREC,
                        preferred_element_type=jnp.float32)
    y2_ref[...] = y2

    y2_hi = y2.astype(jnp.bfloat16)
    ycat_ref[...] = jnp.concatenate(
        [y2_hi, (y2 - y2_hi.astype(jnp.float32)).astype(jnp.bfloat16)],
        axis=1)
    z_ref[...] = jnp.zeros((_N, 2 * _H), jnp.float32)

    def p3(k, carry):
        z_ref[...] += jax.lax.dot_general(
            abf_ref[pl.ds(k * _R, _R), :], ycat_ref[pl.ds(k * _R, _R), :],
            _DN_T, preferred_element_type=jnp.float32)
        return carry

    jax.lax.fori_loop(0, _C, p3, 0)

    z = z_ref[:, :_H] + z_ref[:, _H:]
    x2 = jax.nn.relu(dinv * (z + y2_ref[...]) + b2_ref[...])
    pooled = jnp.sum(x2, axis=0, keepdims=True)
    out_ref[...] = jnp.dot(pooled, wfc_ref[...], precision=_PREC,
                           preferred_element_type=jnp.float32) + bfc_ref[...]


def kernel(adj_matrix, W1, b1, W2, b2, Wfc, bfc):
    return pl.pallas_call(
        _gnn_kernel,
        out_shape=jax.ShapeDtypeStruct((1, Wfc.shape[1]), jnp.float32),
        in_specs=[
            pl.BlockSpec(memory_space=pl.ANY),
            pl.BlockSpec(memory_space=pltpu.MemorySpace.VMEM),
            pl.BlockSpec(memory_space=pltpu.MemorySpace.VMEM),
            pl.BlockSpec(memory_space=pltpu.MemorySpace.VMEM),
            pl.BlockSpec(memory_space=pltpu.MemorySpace.VMEM),
            pl.BlockSpec(memory_space=pltpu.MemorySpace.VMEM),
            pl.BlockSpec(memory_space=pltpu.MemorySpace.VMEM),
        ],
        scratch_shapes=[
            pltpu.VMEM((1, _N), jnp.float32),
            pltpu.VMEM((_N, 1), jnp.float32),
            pltpu.VMEM((_N, _H), jnp.float32),
            pltpu.VMEM((_N, 2 * _H), jnp.float32),
            pltpu.VMEM((_N, 2 * _H), jnp.bfloat16),
            pltpu.VMEM((_N, _N), jnp.bfloat16),
            pltpu.VMEM((2, _R, _N), jnp.int32),
            pltpu.SemaphoreType.DMA((2,)),
        ],
    )(adj_matrix, W1, b1.reshape(1, -1), W2, b2.reshape(1, -1), Wfc,
      bfc.reshape(1, -1))


# speedup vs baseline: 1.0676x; 1.0676x over previous
import jax
import jax.numpy as jnp
from jax.experimental import pallas as pl
from jax.experimental.pallas import tpu as pltpu

_N = 2048
_H = 32
_R = 256
_C = _N // _R
_PREC = jax.lax.Precision.HIGHEST
_DN_T = (((0,), (0,)), ((), ()))


def _gnn_kernel(adj_ref, w1_ref, b1_ref, w2_ref, b2_ref, wfc_ref, bfc_ref,
                out_ref, row_ref, dcol_ref, y2_ref, z_ref, ycat_ref, abf_ref):
    row_ref[...] = jnp.zeros((1, _N), jnp.float32)

    def p1(k, carry):
        af = (adj_ref[pl.ds(k * _R, _R), :] > 0).astype(jnp.float32)
        abf_ref[pl.ds(k * _R, _R), :] = af.astype(jnp.bfloat16)
        row_ref[...] += jnp.sum(af, axis=0, keepdims=True)
        return carry

    jax.lax.fori_loop(0, _C, p1, 0)
    dinv_row = 1.0 / jnp.sqrt(row_ref[...] + 1.0)
    dcol_ref[...] = jnp.reshape(dinv_row, (_N, 1))

    row_ref[...] = jnp.zeros((1, _N), jnp.float32)

    def p2(k, carry):
        d = dcol_ref[pl.ds(k * _R, _R), :]
        af = abf_ref[pl.ds(k * _R, _R), :].astype(jnp.float32)
        row_ref[...] += jnp.sum(af * d, axis=0, keepdims=True)
        return carry

    jax.lax.fori_loop(0, _C, p2, 0)

    alpha_row = dinv_row * (row_ref[...] + dinv_row)
    alpha = jnp.reshape(alpha_row, (_N, 1))
    dinv = dcol_ref[...]
    x1 = jax.nn.relu(alpha * w1_ref[...] + b1_ref[...])
    y2 = dinv * jnp.dot(x1, w2_ref[...], precision=_PREC,
                        preferred_element_type=jnp.float32)
    y2_ref[...] = y2

    y2_hi = y2.astype(jnp.bfloat16)
    ycat_ref[...] = jnp.concatenate(
        [y2_hi, (y2 - y2_hi.astype(jnp.float32)).astype(jnp.bfloat16)],
        axis=1)
    z_ref[...] = jax.lax.dot_general(abf_ref[...], ycat_ref[...], _DN_T,
                                     preferred_element_type=jnp.float32)

    z = z_ref[:, :_H] + z_ref[:, _H:]
    x2 = jax.nn.relu(dinv * (z + y2_ref[...]) + b2_ref[...])
    pooled = jnp.sum(x2, axis=0, keepdims=True)
    out_ref[...] = jnp.dot(pooled, wfc_ref[...], precision=_PREC,
                           preferred_element_type=jnp.float32) + bfc_ref[...]


def kernel(adj_matrix, W1, b1, W2, b2, Wfc, bfc):
    return pl.pallas_call(
        _gnn_kernel,
        out_shape=jax.ShapeDtypeStruct((1, Wfc.shape[1]), jnp.float32),
        scratch_shapes=[
            pltpu.VMEM((1, _N), jnp.float32),
            pltpu.VMEM((_N, 1), jnp.float32),
            pltpu.VMEM((_N, _H), jnp.float32),
            pltpu.VMEM((_N, 2 * _H), jnp.float32),
            pltpu.VMEM((_N, 2 * _H), jnp.bfloat16),
            pltpu.VMEM((_N, _N), jnp.bfloat16),
        ],
    )(adj_matrix, W1, b1.reshape(1, -1), W2, b2.reshape(1, -1), Wfc,
      bfc.reshape(1, -1))
